# baseline (speedup 1.0000x reference)
import jax
import jax.numpy as jnp
from jax.experimental import pallas as pl
from jax.experimental.pallas import tpu as pltpu

_NEG_INF = float("-inf")


def _lrelu(x):
    return jnp.where(x >= 0, x, 0.01 * x)


def _dotT(a, w):
    return jax.lax.dot_general(a, w, (((1,), (1,)), ((), ())),
                               preferred_element_type=jnp.float32)


def _dot(a, b):
    return jax.lax.dot_general(a, b, (((1,), (0,)), ((), ())),
                               preferred_element_type=jnp.float32)


def _dotTA(a, b):
    return jax.lax.dot_general(a, b, (((0,), (0,)), ((), ())),
                               preferred_element_type=jnp.float32)


def _rowsum_as_row(a):
    ones = jnp.ones((1, a.shape[1]), jnp.float32)
    return _dotT(ones, a)


def _make_kernel(n, c, hdim, R, T):
    def _kernel(x_ref, cm_ref, mv_ref, wps_ref, bps_ref, wpsb_ref, bpsb_ref,
                wpsf_ref, bpsf_ref, whs_ref, bhs_ref, whsb_ref, bhsb_ref,
                whsf_ref, bhsf_ref, windi_ref, bindi_ref, wout_ref, bout_ref,
                pred_ref, h_s, outps_s, hid2_s, colsum_s, diag_s, hid2b_s,
                ryn_s, acol_s):
        i = pl.program_id(0)

        @pl.when(i == 0)
        def _stage_a():
            x = x_ref[:]
            cm = cm_ref[:]
            mv = mv_ref[:]
            s2c = cm * mv
            colsum = jnp.sum(s2c, axis=0, keepdims=True)
            s2c = s2c / (colsum * cm + 1.0)
            hidden = _dotTA(s2c, x)
            rs_col = _dot(hidden, jnp.ones((hdim, 1), jnp.float32))
            keep1c = jnp.where(rs_col == 0.0, 0.0, 1.0)
            rs_row = _rowsum_as_row(hidden)
            logits = _dotT(x, hidden)
            m0 = jnp.max(logits, axis=0, keepdims=True)
            e0 = jnp.exp(logits - m0)
            s0 = _dotTA(e0, jnp.ones((n, 1), jnp.float32))
            hidden = _dotTA(e0, x) * (keep1c / s0)
            sx = jnp.sum(x * x, axis=1, keepdims=True)
            sy = _rowsum_as_row(hidden * hidden)
            rxn = jnp.where(sx == 0.0, 0.0, 1.0 / jnp.sqrt(sx))
            ryn = jnp.where(sy == 0.0, 0.0, 1.0 / jnp.sqrt(sy))
            acol = jnp.where(rs_row == 0.0, _NEG_INF, 0.0)
            xy = _dotT(x, hidden)
            c2s = xy * rxn * ryn + acol
            m1 = jnp.max(c2s, axis=1, keepdims=True)
            e1 = jnp.exp(c2s - m1)
            s1 = jnp.sum(e1, axis=1, keepdims=True)
            ps = _dot(e1, hidden) / s1
            ps = _dotT(ps, wps_ref[:]) + bps_ref[:]
            psb = _dotT(ps, wpsb_ref[:]) + bpsb_ref[:]
            outps_s[:] = _lrelu(_dotT(ps, wpsf_ref[:]) + bpsf_ref[:])
            h_s[:] = x - psb
            hid2_s[:] = jnp.zeros_like(hid2_s)
            colsum_s[:] = jnp.zeros_like(colsum_s)

        @pl.when((i >= 1) & (i <= T))
        def _stage_b1():
            j = i - 1
            sl = pl.ds(j * R, R)
            h = h_s[:]
            ht = h_s[sl, :]
            sx = jnp.sum(ht * ht, axis=1, keepdims=True)
            sy = _rowsum_as_row(h * h)
            rxn = jnp.where(sx == 0.0, 0.0, 1.0 / jnp.sqrt(sx))
            ryn = jnp.where(sy == 0.0, 0.0, 1.0 / jnp.sqrt(sy))
            xy = _dotT(ht, h)
            sim = xy * rxn * ryn
            dvals = sx * rxn * rxn
            work = sim
            for _ in range(4):
                m = jnp.max(work, axis=1, keepdims=True)
                d = work - m
                work = jnp.where(d == 0.0, _NEG_INF, work)
            masked = jnp.where(work == _NEG_INF, sim, 0.0)
            colsum_s[:] = colsum_s[:] + _dotTA(
                masked, jnp.ones((R, 1), jnp.float32))
            hid2_s[:] = hid2_s[:] + _dotTA(masked, ht)
            colsum_s[sl, :] = colsum_s[sl, :] - dvals
            hid2_s[sl, :] = hid2_s[sl, :] - dvals * ht
            diag_s[sl, :] = dvals

        @pl.when(i == T + 1)
        def _fixup():
            h = h_s[:]
            dvec = jnp.where(colsum_s[:] != 0.0, diag_s[:], 0.0)
            hid2 = hid2_s[:] + dvec * h
            rs_col = _dot(hid2, jnp.ones((hdim, 1), jnp.float32))
            keep2c = jnp.where(rs_col == 0.0, 0.0, 1.0)
            hid2 = hid2 * keep2c
            hid2b_s[:] = hid2.astype(jnp.bfloat16)
            rs_row = _rowsum_as_row(hid2)
            sy2 = _rowsum_as_row(hid2 * hid2)
            ryn_s[:] = jnp.where(sy2 == 0.0, 0.0, 1.0 / jnp.sqrt(sy2))
            acol_s[:] = jnp.where(rs_row == 0.0, _NEG_INF, 0.0)

        @pl.when(i >= T + 1)
        def _stage_b2():
            j = i - (T + 1)
            sl = pl.ds(j * R, R)
            ht = h_s[sl, :]
            hid2b = hid2b_s[:]
            sx = jnp.sum(ht * ht, axis=1, keepdims=True)
            rxn = jnp.where(sx == 0.0, 0.0, 1.0 / jnp.sqrt(sx))
            xy = _dotT(ht.astype(jnp.bfloat16), hid2b)
            c2s = xy * rxn * ryn_s[:] + acol_s[:]
            m = jnp.max(c2s, axis=1, keepdims=True)
            e = jnp.exp(c2s - m)
            s1 = jnp.sum(e, axis=1, keepdims=True)
            hsi = _dot(e.astype(jnp.bfloat16), hid2b) / s1
            hs = _dotT(hsi, whs_ref[:]) + bhs_ref[:]
            hsb = _dotT(hs, whsb_ref[:]) + bhsb_ref[:]
            ouths = _lrelu(_dotT(hs, whsf_ref[:]) + bhsf_ref[:])
            indiv = ht - hsb
            outind = _lrelu(_dotT(indiv, windi_ref[:]) + bindi_ref[:])
            alli = outps_s[sl, :] + ouths + outind
            pred_ref[:] = (jnp.sum(alli * wout_ref[:], axis=1, keepdims=True)
                           + bout_ref[0, 0])

    return _kernel


def kernel(x_hidden, concept_matrix, market_value, W_ps, b_ps, W_hs, b_hs,
           W_ps_fore, b_ps_fore, W_hs_fore, b_hs_fore, W_ps_back, b_ps_back,
           W_hs_back, b_hs_back, W_indi, b_indi, W_out_ps, b_out_ps,
           W_out_hs, b_out_hs, W_out_indi, b_out_indi, W_out, b_out):
    n, hdim = x_hidden.shape
    c = concept_matrix.shape[1]
    f32 = jnp.float32
    mv = market_value.reshape(n, 1)

    R = 128
    T = n // R

    full = lambda shp: pl.BlockSpec(shp, lambda i: tuple(0 for _ in shp))

    pred = pl.pallas_call(
        _make_kernel(n, c, hdim, R, T),
        grid=(1 + 2 * T,),
        in_specs=[full((n, hdim)), full((n, c)), full((n, 1)),
                  full((hdim, hdim)), full((1, hdim)),
                  full((hdim, hdim)), full((1, hdim)),
                  full((hdim, hdim)), full((1, hdim)),
                  full((hdim, hdim)), full((1, hdim)),
                  full((hdim, hdim)), full((1, hdim)),
                  full((hdim, hdim)), full((1, hdim)),
                  full((hdim, hdim)), full((1, hdim)),
                  full((1, hdim)), full((1, 1))],
        out_specs=pl.BlockSpec((R, 1),
                               lambda i: (jnp.maximum(i - (T + 1), 0), 0)),
        out_shape=jax.ShapeDtypeStruct((n, 1), f32),
        scratch_shapes=[pltpu.VMEM((n, hdim), f32),
                        pltpu.VMEM((n, hdim), f32),
                        pltpu.VMEM((n, hdim), f32),
                        pltpu.VMEM((n, 1), f32),
                        pltpu.VMEM((n, 1), f32),
                        pltpu.VMEM((n, hdim), jnp.bfloat16),
                        pltpu.VMEM((1, n), f32),
                        pltpu.VMEM((1, n), f32)],
    )(x_hidden, concept_matrix, mv,
      W_ps, b_ps.reshape(1, hdim), W_ps_back, b_ps_back.reshape(1, hdim),
      W_ps_fore, b_ps_fore.reshape(1, hdim),
      W_hs, b_hs.reshape(1, hdim), W_hs_back, b_hs_back.reshape(1, hdim),
      W_hs_fore, b_hs_fore.reshape(1, hdim), W_indi, b_indi.reshape(1, hdim),
      W_out, b_out.reshape(1, 1))

    return pred.reshape(n)


# speedup vs baseline: 1.1026x; 1.1026x over previous
import jax
import jax.numpy as jnp
from jax.experimental import pallas as pl
from jax.experimental.pallas import tpu as pltpu

_NEG_INF = float("-inf")


def _lrelu(x):
    return jnp.where(x >= 0, x, 0.01 * x)


def _dotT(a, w):
    return jax.lax.dot_general(a, w, (((1,), (1,)), ((), ())),
                               preferred_element_type=jnp.float32)


def _dot(a, b):
    return jax.lax.dot_general(a, b, (((1,), (0,)), ((), ())),
                               preferred_element_type=jnp.float32)


def _dotTA(a, b):
    return jax.lax.dot_general(a, b, (((0,), (0,)), ((), ())),
                               preferred_element_type=jnp.float32)


def _rowsum_as_row(a):
    ones = jnp.ones((1, a.shape[1]), jnp.float32)
    return _dotT(ones, a)


def _make_kernel(n, c, hdim, R, T):
    def _kernel(x_ref, cm_ref, mv_ref, wps_ref, bps_ref, wpsb_ref, bpsb_ref,
                wpsf_ref, bpsf_ref, whs_ref, bhs_ref, whsb_ref, bhsb_ref,
                whsf_ref, bhsf_ref, windi_ref, bindi_ref, wout_ref, bout_ref,
                pred_ref, h_s, outps_s, hid2_s, colsum_s, diag_s, hid2b_s,
                ryn_s, acol_s):
        i = pl.program_id(0)

        @pl.when(i == 0)
        def _stage_a():
            x = x_ref[:]
            cm = cm_ref[:]
            mv = mv_ref[:]
            s2c = cm * mv
            colsum = jnp.sum(s2c, axis=0, keepdims=True)
            s2c = s2c / (colsum * cm + 1.0)
            hidden = _dotTA(s2c, x)
            rs_col = _dot(hidden, jnp.ones((hdim, 1), jnp.float32))
            keep1c = jnp.where(rs_col == 0.0, 0.0, 1.0)
            rs_row = _rowsum_as_row(hidden)
            logits = _dotT(x, hidden)
            m0 = jnp.max(logits, axis=0, keepdims=True)
            e0 = jnp.exp(logits - m0)
            s0 = _dotTA(e0, jnp.ones((n, 1), jnp.float32))
            hidden = _dotTA(e0, x) * (keep1c / s0)
            sx = jnp.sum(x * x, axis=1, keepdims=True)
            sy = _rowsum_as_row(hidden * hidden)
            rxn = jnp.where(sx == 0.0, 0.0, 1.0 / jnp.sqrt(sx))
            ryn = jnp.where(sy == 0.0, 0.0, 1.0 / jnp.sqrt(sy))
            acol = jnp.where(rs_row == 0.0, _NEG_INF, 0.0)
            xy = _dotT(x, hidden)
            c2s = xy * rxn * ryn + acol
            m1 = jnp.max(c2s, axis=1, keepdims=True)
            e1 = jnp.exp(c2s - m1)
            s1 = jnp.sum(e1, axis=1, keepdims=True)
            ps = _dot(e1, hidden) / s1
            ps = _dotT(ps, wps_ref[:]) + bps_ref[:]
            psb = _dotT(ps, wpsb_ref[:]) + bpsb_ref[:]
            outps_s[:] = _lrelu(_dotT(ps, wpsf_ref[:]) + bpsf_ref[:])
            h_s[:] = x - psb
            hid2_s[:] = jnp.zeros_like(hid2_s)
            colsum_s[:] = jnp.zeros_like(colsum_s)

        @pl.when((i >= 1) & (i <= T))
        def _stage_b1():
            j = i - 1
            sl = pl.ds(j * R, R)
            h = h_s[:]
            ht = h_s[sl, :]
            sx = jnp.sum(ht * ht, axis=1, keepdims=True)
            sy = _rowsum_as_row(h * h)
            rxn = jnp.where(sx == 0.0, 0.0, 1.0 / jnp.sqrt(sx))
            ryn = jnp.where(sy == 0.0, 0.0, 1.0 / jnp.sqrt(sy))
            xy = _dotT(ht, h)
            sim = xy * rxn * ryn
            dvals = sx * rxn * rxn
            work = sim
            for _ in range(4):
                m = jnp.max(work, axis=1, keepdims=True)
                d = work - m
                work = jnp.where(d == 0.0, _NEG_INF, work)
            masked = jnp.where(work == _NEG_INF, sim, 0.0)
            colsum_s[:] = colsum_s[:] + _dotTA(
                masked, jnp.ones((R, 1), jnp.float32))
            hid2_s[:] = hid2_s[:] + _dotTA(masked, ht)
            colsum_s[sl, :] = colsum_s[sl, :] - dvals
            hid2_s[sl, :] = hid2_s[sl, :] - dvals * ht
            diag_s[sl, :] = dvals

        @pl.when(i == T + 1)
        def _fixup():
            h = h_s[:]
            dvec = jnp.where(colsum_s[:] != 0.0, diag_s[:], 0.0)
            hid2 = hid2_s[:] + dvec * h
            rs_col = _dot(hid2, jnp.ones((hdim, 1), jnp.float32))
            keep2c = jnp.where(rs_col == 0.0, 0.0, 1.0)
            hid2 = hid2 * keep2c
            hid2b_s[:] = hid2.astype(jnp.bfloat16)
            rs_row = _rowsum_as_row(hid2)
            sy2 = _rowsum_as_row(hid2 * hid2)
            ryn_s[:] = jnp.where(sy2 == 0.0, 0.0, 1.0 / jnp.sqrt(sy2))
            acol_s[:] = jnp.where(rs_row == 0.0, _NEG_INF, 0.0)

        @pl.when(i >= T + 1)
        def _stage_b2():
            j = i - (T + 1)
            sl = pl.ds(j * R, R)
            ht = h_s[sl, :]
            hid2b = hid2b_s[:]
            sx = jnp.sum(ht * ht, axis=1, keepdims=True)
            rxn = jnp.where(sx == 0.0, 0.0, 1.0 / jnp.sqrt(sx))
            xy = _dotT(ht.astype(jnp.bfloat16), hid2b)
            c2s = xy * rxn * ryn_s[:] + acol_s[:]
            m = jnp.max(c2s, axis=1, keepdims=True)
            e = jnp.exp(c2s - m)
            s1 = jnp.sum(e, axis=1, keepdims=True)
            hsi = _dot(e.astype(jnp.bfloat16), hid2b) / s1
            hs = _dotT(hsi, whs_ref[:]) + bhs_ref[:]
            hsb = _dotT(hs, whsb_ref[:]) + bhsb_ref[:]
            ouths = _lrelu(_dotT(hs, whsf_ref[:]) + bhsf_ref[:])
            indiv = ht - hsb
            outind = _lrelu(_dotT(indiv, windi_ref[:]) + bindi_ref[:])
            alli = outps_s[sl, :] + ouths + outind
            pred_ref[:] = (jnp.sum(alli * wout_ref[:], axis=1, keepdims=True)
                           + bout_ref[0, 0])

    return _kernel


def kernel(x_hidden, concept_matrix, market_value, W_ps, b_ps, W_hs, b_hs,
           W_ps_fore, b_ps_fore, W_hs_fore, b_hs_fore, W_ps_back, b_ps_back,
           W_hs_back, b_hs_back, W_indi, b_indi, W_out_ps, b_out_ps,
           W_out_hs, b_out_hs, W_out_indi, b_out_indi, W_out, b_out):
    n, hdim = x_hidden.shape
    c = concept_matrix.shape[1]
    f32 = jnp.float32
    mv = market_value.reshape(n, 1)

    R = 256
    T = n // R

    full = lambda shp: pl.BlockSpec(shp, lambda i: tuple(0 for _ in shp))

    pred = pl.pallas_call(
        _make_kernel(n, c, hdim, R, T),
        grid=(1 + 2 * T,),
        in_specs=[full((n, hdim)), full((n, c)), full((n, 1)),
                  full((hdim, hdim)), full((1, hdim)),
                  full((hdim, hdim)), full((1, hdim)),
                  full((hdim, hdim)), full((1, hdim)),
                  full((hdim, hdim)), full((1, hdim)),
                  full((hdim, hdim)), full((1, hdim)),
                  full((hdim, hdim)), full((1, hdim)),
                  full((hdim, hdim)), full((1, hdim)),
                  full((1, hdim)), full((1, 1))],
        out_specs=pl.BlockSpec((R, 1),
                               lambda i: (jnp.maximum(i - (T + 1), 0), 0)),
        out_shape=jax.ShapeDtypeStruct((n, 1), f32),
        scratch_shapes=[pltpu.VMEM((n, hdim), f32),
                        pltpu.VMEM((n, hdim), f32),
                        pltpu.VMEM((n, hdim), f32),
                        pltpu.VMEM((n, 1), f32),
                        pltpu.VMEM((n, 1), f32),
                        pltpu.VMEM((n, hdim), jnp.bfloat16),
                        pltpu.VMEM((1, n), f32),
                        pltpu.VMEM((1, n), f32)],
    )(x_hidden, concept_matrix, mv,
      W_ps, b_ps.reshape(1, hdim), W_ps_back, b_ps_back.reshape(1, hdim),
      W_ps_fore, b_ps_fore.reshape(1, hdim),
      W_hs, b_hs.reshape(1, hdim), W_hs_back, b_hs_back.reshape(1, hdim),
      W_hs_fore, b_hs_fore.reshape(1, hdim), W_indi, b_indi.reshape(1, hdim),
      W_out, b_out.reshape(1, 1))

    return pred.reshape(n)


# speedup vs baseline: 1.1292x; 1.0241x over previous
import jax
import jax.numpy as jnp
from jax.experimental import pallas as pl
from jax.experimental.pallas import tpu as pltpu

_NEG_INF = float("-inf")


def _lrelu(x):
    return jnp.where(x >= 0, x, 0.01 * x)


def _dotT(a, w):
    return jax.lax.dot_general(a, w, (((1,), (1,)), ((), ())),
                               preferred_element_type=jnp.float32)


def _dot(a, b):
    return jax.lax.dot_general(a, b, (((1,), (0,)), ((), ())),
                               preferred_element_type=jnp.float32)


def _dotTA(a, b):
    return jax.lax.dot_general(a, b, (((0,), (0,)), ((), ())),
                               preferred_element_type=jnp.float32)


def _rowsum_as_row(a):
    ones = jnp.ones((1, a.shape[1]), jnp.float32)
    return _dotT(ones, a)


def _make_kernel(n, c, hdim, R, T):
    def _kernel(x_ref, cm_ref, mv_ref, wps_ref, bps_ref, wpsb_ref, bpsb_ref,
                wpsf_ref, bpsf_ref, whs_ref, bhs_ref, whsb_ref, bhsb_ref,
                whsf_ref, bhsf_ref, windi_ref, bindi_ref, wout_ref, bout_ref,
                pred_ref, h_s, outps_s, hid2_s, colsum_s, diag_s, hid2b_s,
                ryn_s, acol_s, ryn1_s):
        i = pl.program_id(0)

        @pl.when(i == 0)
        def _stage_a():
            x = x_ref[:]
            cm = cm_ref[:]
            mv = mv_ref[:]
            s2c = cm * mv
            colsum = jnp.sum(s2c, axis=0, keepdims=True)
            s2c = s2c / (colsum * cm + 1.0)
            hidden = _dotTA(s2c, x)
            rs_col = _dot(hidden, jnp.ones((hdim, 1), jnp.float32))
            keep1c = jnp.where(rs_col == 0.0, 0.0, 1.0)
            rs_row = _rowsum_as_row(hidden)
            logits = _dotT(x, hidden)
            m0 = jnp.max(logits, axis=0, keepdims=True)
            e0 = jnp.exp(logits - m0)
            s0 = _dotTA(e0, jnp.ones((n, 1), jnp.float32))
            hidden = _dotTA(e0, x) * (keep1c / s0)
            sx = jnp.sum(x * x, axis=1, keepdims=True)
            sy = _rowsum_as_row(hidden * hidden)
            rxn = jnp.where(sx == 0.0, 0.0, 1.0 / jnp.sqrt(sx))
            ryn = jnp.where(sy == 0.0, 0.0, 1.0 / jnp.sqrt(sy))
            acol = jnp.where(rs_row == 0.0, _NEG_INF, 0.0)
            xy = _dotT(x, hidden)
            c2s = xy * rxn * ryn + acol
            m1 = jnp.max(c2s, axis=1, keepdims=True)
            e1 = jnp.exp(c2s - m1)
            s1 = jnp.sum(e1, axis=1, keepdims=True)
            ps = _dot(e1, hidden) / s1
            ps = _dotT(ps, wps_ref[:]) + bps_ref[:]
            psb = _dotT(ps, wpsb_ref[:]) + bpsb_ref[:]
            outps_s[:] = _lrelu(_dotT(ps, wpsf_ref[:]) + bpsf_ref[:])
            h = x - psb
            h_s[:] = h
            sy1 = _rowsum_as_row(h * h)
            ryn1_s[:] = jnp.where(sy1 == 0.0, 0.0, 1.0 / jnp.sqrt(sy1))
            hid2_s[:] = jnp.zeros_like(hid2_s)
            colsum_s[:] = jnp.zeros_like(colsum_s)

        @pl.when((i >= 1) & (i <= T))
        def _stage_b1():
            j = i - 1
            sl = pl.ds(j * R, R)
            h = h_s[:]
            ht = h_s[sl, :]
            sx = jnp.sum(ht * ht, axis=1, keepdims=True)
            rxn = jnp.where(sx == 0.0, 0.0, 1.0 / jnp.sqrt(sx))
            ryn = ryn1_s[:]
            xy = _dotT(ht, h)
            sim = xy * rxn * ryn
            dvals = sx * rxn * rxn
            work = sim
            for _ in range(4):
                m = jnp.max(work, axis=1, keepdims=True)
                d = work - m
                work = jnp.where(d == 0.0, _NEG_INF, work)
            masked = jnp.where(work == _NEG_INF, sim, 0.0)
            colsum_s[:] = colsum_s[:] + _dotTA(
                masked, jnp.ones((R, 1), jnp.float32))
            hid2_s[:] = hid2_s[:] + _dotTA(masked, ht)
            colsum_s[sl, :] = colsum_s[sl, :] - dvals
            hid2_s[sl, :] = hid2_s[sl, :] - dvals * ht
            diag_s[sl, :] = dvals

        @pl.when(i == T + 1)
        def _fixup():
            h = h_s[:]
            dvec = jnp.where(colsum_s[:] != 0.0, diag_s[:], 0.0)
            hid2 = hid2_s[:] + dvec * h
            rs_col = _dot(hid2, jnp.ones((hdim, 1), jnp.float32))
            keep2c = jnp.where(rs_col == 0.0, 0.0, 1.0)
            hid2 = hid2 * keep2c
            hid2b_s[:] = hid2.astype(jnp.bfloat16)
            rs_row = _rowsum_as_row(hid2)
            sy2 = _rowsum_as_row(hid2 * hid2)
            ryn_s[:] = jnp.where(sy2 == 0.0, 0.0, 1.0 / jnp.sqrt(sy2))
            acol_s[:] = jnp.where(rs_row == 0.0, _NEG_INF, 0.0)

        @pl.when(i >= T + 1)
        def _stage_b2():
            j = i - (T + 1)
            sl = pl.ds(j * R, R)
            ht = h_s[sl, :]
            hid2b = hid2b_s[:]
            sx = jnp.sum(ht * ht, axis=1, keepdims=True)
            rxn = jnp.where(sx == 0.0, 0.0, 1.0 / jnp.sqrt(sx))
            xy = _dotT(ht.astype(jnp.bfloat16), hid2b)
            c2s = xy * rxn * ryn_s[:] + acol_s[:]
            m = jnp.max(c2s, axis=1, keepdims=True)
            e = jnp.exp(c2s - m)
            s1 = jnp.sum(e, axis=1, keepdims=True)
            hsi = _dot(e.astype(jnp.bfloat16), hid2b) / s1
            hs = _dotT(hsi, whs_ref[:]) + bhs_ref[:]
            hsb = _dotT(hs, whsb_ref[:]) + bhsb_ref[:]
            ouths = _lrelu(_dotT(hs, whsf_ref[:]) + bhsf_ref[:])
            indiv = ht - hsb
            outind = _lrelu(_dotT(indiv, windi_ref[:]) + bindi_ref[:])
            alli = outps_s[sl, :] + ouths + outind
            pred_ref[:] = (jnp.sum(alli * wout_ref[:], axis=1, keepdims=True)
                           + bout_ref[0, 0])

    return _kernel


def kernel(x_hidden, concept_matrix, market_value, W_ps, b_ps, W_hs, b_hs,
           W_ps_fore, b_ps_fore, W_hs_fore, b_hs_fore, W_ps_back, b_ps_back,
           W_hs_back, b_hs_back, W_indi, b_indi, W_out_ps, b_out_ps,
           W_out_hs, b_out_hs, W_out_indi, b_out_indi, W_out, b_out):
    n, hdim = x_hidden.shape
    c = concept_matrix.shape[1]
    f32 = jnp.float32
    mv = market_value.reshape(n, 1)

    R = 256
    T = n // R

    full = lambda shp: pl.BlockSpec(shp, lambda i: tuple(0 for _ in shp))

    pred = pl.pallas_call(
        _make_kernel(n, c, hdim, R, T),
        grid=(1 + 2 * T,),
        in_specs=[full((n, hdim)), full((n, c)), full((n, 1)),
                  full((hdim, hdim)), full((1, hdim)),
                  full((hdim, hdim)), full((1, hdim)),
                  full((hdim, hdim)), full((1, hdim)),
                  full((hdim, hdim)), full((1, hdim)),
                  full((hdim, hdim)), full((1, hdim)),
                  full((hdim, hdim)), full((1, hdim)),
                  full((hdim, hdim)), full((1, hdim)),
                  full((1, hdim)), full((1, 1))],
        out_specs=pl.BlockSpec((R, 1),
                               lambda i: (jnp.maximum(i - (T + 1), 0), 0)),
        out_shape=jax.ShapeDtypeStruct((n, 1), f32),
        scratch_shapes=[pltpu.VMEM((n, hdim), f32),
                        pltpu.VMEM((n, hdim), f32),
                        pltpu.VMEM((n, hdim), f32),
                        pltpu.VMEM((n, 1), f32),
                        pltpu.VMEM((n, 1), f32),
                        pltpu.VMEM((n, hdim), jnp.bfloat16),
                        pltpu.VMEM((1, n), f32),
                        pltpu.VMEM((1, n), f32),
                        pltpu.VMEM((1, n), f32)],
    )(x_hidden, concept_matrix, mv,
      W_ps, b_ps.reshape(1, hdim), W_ps_back, b_ps_back.reshape(1, hdim),
      W_ps_fore, b_ps_fore.reshape(1, hdim),
      W_hs, b_hs.reshape(1, hdim), W_hs_back, b_hs_back.reshape(1, hdim),
      W_hs_fore, b_hs_fore.reshape(1, hdim), W_indi, b_indi.reshape(1, hdim),
      W_out, b_out.reshape(1, 1))

    return pred.reshape(n)


# speedup vs baseline: 1.1502x; 1.0186x over previous
import jax
import jax.numpy as jnp
from jax.experimental import pallas as pl
from jax.experimental.pallas import tpu as pltpu

_NEG_INF = float("-inf")


def _lrelu(x):
    return jnp.where(x >= 0, x, 0.01 * x)


def _dotT(a, w):
    return jax.lax.dot_general(a, w, (((1,), (1,)), ((), ())),
                               preferred_element_type=jnp.float32)


def _dot(a, b):
    return jax.lax.dot_general(a, b, (((1,), (0,)), ((), ())),
                               preferred_element_type=jnp.float32)


def _dotTA(a, b):
    return jax.lax.dot_general(a, b, (((0,), (0,)), ((), ())),
                               preferred_element_type=jnp.float32)


def _rowsum_as_row(a):
    ones = jnp.ones((1, a.shape[1]), jnp.float32)
    return _dotT(ones, a)


def _make_kernel(n, c, hdim, R, T):
    def _kernel(x_ref, cm_ref, mv_ref, wps_ref, bps_ref, wpsb_ref, bpsb_ref,
                wpsf_ref, bpsf_ref, whs_ref, bhs_ref, whsb_ref, bhsb_ref,
                whsf_ref, bhsf_ref, windi_ref, bindi_ref, wout_ref, bout_ref,
                pred_ref, h_s, outps_s, hid2_s, colsum_s, diag_s, hid2b_s,
                ryn_s, acol_s, ryn1_s):
        i = pl.program_id(0)

        @pl.when(i == 0)
        def _stage_a():
            x = x_ref[:]
            cm = cm_ref[:]
            mv = mv_ref[:]
            s2c = cm * mv
            colsum = jnp.sum(s2c, axis=0, keepdims=True)
            s2c = s2c / (colsum * cm + 1.0)
            hidden = _dotTA(s2c, x)
            rs_col = _dot(hidden, jnp.ones((hdim, 1), jnp.float32))
            keep1c = jnp.where(rs_col == 0.0, 0.0, 1.0)
            rs_row = _rowsum_as_row(hidden)
            logits = _dotT(x, hidden)
            m0 = jnp.max(logits, axis=0, keepdims=True)
            e0 = jnp.exp(logits - m0)
            s0 = _dotTA(e0, jnp.ones((n, 1), jnp.float32))
            hidden = _dotTA(e0, x) * (keep1c / s0)
            sx = jnp.sum(x * x, axis=1, keepdims=True)
            sy = _rowsum_as_row(hidden * hidden)
            rxn = jnp.where(sx == 0.0, 0.0, 1.0 / jnp.sqrt(sx))
            ryn = jnp.where(sy == 0.0, 0.0, 1.0 / jnp.sqrt(sy))
            acol = jnp.where(rs_row == 0.0, _NEG_INF, 0.0)
            xy = _dotT(x, hidden)
            c2s = xy * rxn * ryn + acol
            m1 = jnp.max(c2s, axis=1, keepdims=True)
            e1 = jnp.exp(c2s - m1)
            s1 = jnp.sum(e1, axis=1, keepdims=True)
            ps = _dot(e1, hidden) / s1
            ps = _dotT(ps, wps_ref[:]) + bps_ref[:]
            psb = _dotT(ps, wpsb_ref[:]) + bpsb_ref[:]
            outps_s[:] = _lrelu(_dotT(ps, wpsf_ref[:]) + bpsf_ref[:])
            h = x - psb
            h_s[:] = h
            sy1 = _rowsum_as_row(h * h)
            ryn1_s[:] = jnp.where(sy1 == 0.0, 0.0, 1.0 / jnp.sqrt(sy1))
            hid2_s[:] = jnp.zeros_like(hid2_s)
            colsum_s[:] = jnp.zeros_like(colsum_s)

        @pl.when((i >= 1) & (i <= T))
        def _stage_b1():
            j = i - 1
            sl = pl.ds(j * R, R)
            h = h_s[:]
            ht = h_s[sl, :]
            sx = jnp.sum(ht * ht, axis=1, keepdims=True)
            rxn = jnp.where(sx == 0.0, 0.0, 1.0 / jnp.sqrt(sx))
            xy = _dotT(ht, h)
            simc = xy * ryn1_s[:]
            dvals = sx * rxn * rxn
            work = simc
            for _ in range(4):
                m = jnp.max(work, axis=1, keepdims=True)
                d = work - m
                work = jnp.where(d == 0.0, _NEG_INF, work)
            masked = jnp.where(work == _NEG_INF, simc, 0.0)
            colsum_s[:] = colsum_s[:] + _dotTA(masked, rxn)
            hid2_s[:] = hid2_s[:] + _dotTA(masked, rxn * ht)
            colsum_s[sl, :] = colsum_s[sl, :] - dvals
            hid2_s[sl, :] = hid2_s[sl, :] - dvals * ht
            diag_s[sl, :] = dvals

        @pl.when(i == T + 1)
        def _fixup():
            h = h_s[:]
            dvec = jnp.where(colsum_s[:] != 0.0, diag_s[:], 0.0)
            hid2 = hid2_s[:] + dvec * h
            rs_col = _dot(hid2, jnp.ones((hdim, 1), jnp.float32))
            keep2c = jnp.where(rs_col == 0.0, 0.0, 1.0)
            hid2 = hid2 * keep2c
            hid2b_s[:] = hid2.astype(jnp.bfloat16)
            rs_row = _rowsum_as_row(hid2)
            sy2 = _rowsum_as_row(hid2 * hid2)
            ryn_s[:] = jnp.where(sy2 == 0.0, 0.0, 1.0 / jnp.sqrt(sy2))
            acol_s[:] = jnp.where(rs_row == 0.0, _NEG_INF, 0.0)

        @pl.when(i >= T + 1)
        def _stage_b2():
            j = i - (T + 1)
            sl = pl.ds(j * R, R)
            ht = h_s[sl, :]
            hid2b = hid2b_s[:]
            sx = jnp.sum(ht * ht, axis=1, keepdims=True)
            rxn = jnp.where(sx == 0.0, 0.0, 1.0 / jnp.sqrt(sx))
            xy = _dotT(ht.astype(jnp.bfloat16), hid2b)
            c2s = xy * rxn * ryn_s[:] + acol_s[:]
            m = jnp.max(c2s, axis=1, keepdims=True)
            e = jnp.exp(c2s - m)
            s1 = jnp.sum(e, axis=1, keepdims=True)
            hsi = _dot(e.astype(jnp.bfloat16), hid2b) / s1
            hs = _dotT(hsi, whs_ref[:]) + bhs_ref[:]
            hsb = _dotT(hs, whsb_ref[:]) + bhsb_ref[:]
            ouths = _lrelu(_dotT(hs, whsf_ref[:]) + bhsf_ref[:])
            indiv = ht - hsb
            outind = _lrelu(_dotT(indiv, windi_ref[:]) + bindi_ref[:])
            alli = outps_s[sl, :] + ouths + outind
            pred_ref[:] = (jnp.sum(alli * wout_ref[:], axis=1, keepdims=True)
                           + bout_ref[0, 0])

    return _kernel


def kernel(x_hidden, concept_matrix, market_value, W_ps, b_ps, W_hs, b_hs,
           W_ps_fore, b_ps_fore, W_hs_fore, b_hs_fore, W_ps_back, b_ps_back,
           W_hs_back, b_hs_back, W_indi, b_indi, W_out_ps, b_out_ps,
           W_out_hs, b_out_hs, W_out_indi, b_out_indi, W_out, b_out):
    n, hdim = x_hidden.shape
    c = concept_matrix.shape[1]
    f32 = jnp.float32
    mv = market_value.reshape(n, 1)

    R = 256
    T = n // R

    full = lambda shp: pl.BlockSpec(shp, lambda i: tuple(0 for _ in shp))

    pred = pl.pallas_call(
        _make_kernel(n, c, hdim, R, T),
        grid=(1 + 2 * T,),
        in_specs=[full((n, hdim)), full((n, c)), full((n, 1)),
                  full((hdim, hdim)), full((1, hdim)),
                  full((hdim, hdim)), full((1, hdim)),
                  full((hdim, hdim)), full((1, hdim)),
                  full((hdim, hdim)), full((1, hdim)),
                  full((hdim, hdim)), full((1, hdim)),
                  full((hdim, hdim)), full((1, hdim)),
                  full((hdim, hdim)), full((1, hdim)),
                  full((1, hdim)), full((1, 1))],
        out_specs=pl.BlockSpec((R, 1),
                               lambda i: (jnp.maximum(i - (T + 1), 0), 0)),
        out_shape=jax.ShapeDtypeStruct((n, 1), f32),
        scratch_shapes=[pltpu.VMEM((n, hdim), f32),
                        pltpu.VMEM((n, hdim), f32),
                        pltpu.VMEM((n, hdim), f32),
                        pltpu.VMEM((n, 1), f32),
                        pltpu.VMEM((n, 1), f32),
                        pltpu.VMEM((n, hdim), jnp.bfloat16),
                        pltpu.VMEM((1, n), f32),
                        pltpu.VMEM((1, n), f32),
                        pltpu.VMEM((1, n), f32)],
    )(x_hidden, concept_matrix, mv,
      W_ps, b_ps.reshape(1, hdim), W_ps_back, b_ps_back.reshape(1, hdim),
      W_ps_fore, b_ps_fore.reshape(1, hdim),
      W_hs, b_hs.reshape(1, hdim), W_hs_back, b_hs_back.reshape(1, hdim),
      W_hs_fore, b_hs_fore.reshape(1, hdim), W_indi, b_indi.reshape(1, hdim),
      W_out, b_out.reshape(1, 1))

    return pred.reshape(n)
